# Initial kernel scaffold; baseline (speedup 1.0000x reference)
import jax
import jax.numpy as jnp
from jax.experimental import pallas as pl


def kernel(x, B, gamma, beta):
    raise NotImplementedError("write your pallas kernel here")


# speedup vs baseline: 1.1931x; 1.1931x over previous
import functools

import jax
import jax.numpy as jnp
from jax import lax
from jax.experimental import pallas as pl
from jax.experimental.pallas import tpu as pltpu
from jax.experimental.pallas import tpu_sc as plsc

_ACTIONS = 100
_EMB = 32
_DELTA = 5000
_LANES = 16

_NC = 2
_NS = 16
_NW = _NC * _NS

_CHUNK = 512
_GATHER = 128


def _prelude_body(x_ref, gamma_ref, beta_ref, idxlo_ref, idxhi_ref,
                  wl_ref, wh_ref):
    x = x_ref[...]
    mean = jnp.mean(x, axis=0, keepdims=True)
    var = jnp.mean((x - mean) ** 2, axis=0, keepdims=True)
    xn = (x - mean) / jnp.sqrt(var + 1e-5) * gamma_ref[...] + beta_ref[...]
    xt = jnp.tanh(xn)
    xc = jnp.clip(xt, -1.0 + 1e-5, 1.0 - 1e-5)
    ind = lax.broadcasted_iota(jnp.int32, x.shape, 1)
    xl = jnp.floor(xc * _DELTA)
    xh = jnp.floor(xc * _DELTA + 1)
    xli = _ACTIONS * (xl.astype(jnp.int32) + _DELTA) + ind
    xhi = _ACTIONS * (xh.astype(jnp.int32) + _DELTA) + ind
    d = 1.0 / _DELTA
    wh = (xc - xl / _DELTA) / d
    wl = (xh / _DELTA - xc) / d
    idxlo_ref[...] = xli
    idxhi_ref[...] = xhi
    wl_ref[...] = wl
    wh_ref[...] = wh


def _sc_body(idxlo_hbm, idxhi_hbm, wl_hbm, wh_hbm, table_hbm, out_hbm,
             idxlo_v, idxhi_v, wl_v, wh_v, bl_v, bh_v, out_v, sem1, sem2):
    n = out_hbm.shape[0]
    per_w = n // _NW
    nchunks = per_w // _CHUNK
    wid = lax.axis_index("s") * _NC + lax.axis_index("c")

    def chunk(g, carry):
        base = wid * per_w + g * _CHUNK
        pltpu.sync_copy(idxlo_hbm.at[pl.ds(base, _CHUNK)], idxlo_v)
        pltpu.sync_copy(idxhi_hbm.at[pl.ds(base, _CHUNK)], idxhi_v)
        copies = []
        for j in range(_CHUNK // _GATHER):
            sl = pl.ds(j * _GATHER, _GATHER)
            copies.append(pltpu.async_copy(
                table_hbm.at[idxlo_v.at[sl]], bl_v.at[sl], sem1))
            copies.append(pltpu.async_copy(
                table_hbm.at[idxhi_v.at[sl]], bh_v.at[sl], sem2))
        pltpu.sync_copy(wl_hbm.at[pl.ds(base, _CHUNK)], wl_v)
        pltpu.sync_copy(wh_hbm.at[pl.ds(base, _CHUNK)], wh_v)
        for cp in copies:
            cp.wait()

        def elem(e, c):
            esp = jnp.full((_LANES,), e, dtype=jnp.int32)
            wlb = plsc.load_gather(wl_v, [esp])
            whb = plsc.load_gather(wh_v, [esp])
            for h in range(_EMB // _LANES):
                s = pl.ds(h * _LANES, _LANES)
                out_v[e, s] = bl_v[e, s] * wlb + bh_v[e, s] * whb
            return c

        lax.fori_loop(0, _CHUNK, elem, 0)
        pltpu.sync_copy(out_v, out_hbm.at[pl.ds(base, _CHUNK)])
        return carry

    lax.fori_loop(0, nchunks, chunk, 0)


@jax.jit
def kernel(x, B, gamma, beta):
    n, actions = x.shape
    idxlo, idxhi, wl, wh = pl.pallas_call(
        _prelude_body,
        out_shape=[
            jax.ShapeDtypeStruct((n, actions), jnp.int32),
            jax.ShapeDtypeStruct((n, actions), jnp.int32),
            jax.ShapeDtypeStruct((n, actions), jnp.float32),
            jax.ShapeDtypeStruct((n, actions), jnp.float32),
        ],
    )(x, gamma.reshape(1, actions), beta.reshape(1, actions))

    total = n * actions
    mesh = plsc.VectorSubcoreMesh(core_axis_name="c", subcore_axis_name="s")
    sc = pl.kernel(
        _sc_body,
        out_type=jax.ShapeDtypeStruct((total, _EMB), jnp.float32),
        mesh=mesh,
        compiler_params=pltpu.CompilerParams(
            needs_layout_passes=False, use_tc_tiling_on_sc=False),
        scratch_types=[
            pltpu.VMEM((_CHUNK,), jnp.int32),
            pltpu.VMEM((_CHUNK,), jnp.int32),
            pltpu.VMEM((_CHUNK,), jnp.float32),
            pltpu.VMEM((_CHUNK,), jnp.float32),
            pltpu.VMEM((_CHUNK, _EMB), jnp.float32),
            pltpu.VMEM((_CHUNK, _EMB), jnp.float32),
            pltpu.VMEM((_CHUNK, _EMB), jnp.float32),
            pltpu.SemaphoreType.DMA,
            pltpu.SemaphoreType.DMA,
        ],
    )
    out = sc(idxlo.reshape(total), idxhi.reshape(total),
             wl.reshape(total), wh.reshape(total), B)
    return out.reshape(n, actions, _EMB)
